# baseline (speedup 1.0000x reference)
import functools

import jax
import jax.numpy as jnp
from jax.experimental import pallas as pl
from jax.experimental.pallas import tpu as pltpu

EPS = 1e-06
LANES = 128
TOPK = 5
BK = 4096


def _ce(a, b):
    return jnp.maximum(a, b), jnp.minimum(a, b)


def _oem(a, b, keep=TOPK):
    if not a:
        return b[:keep]
    if not b:
        return a[:keep]
    if len(a) == 1 and len(b) == 1:
        hi, lo = _ce(a[0], b[0])
        return [hi, lo][:keep]
    ev = _oem(a[0::2], b[0::2], keep)
    od = _oem(a[1::2], b[1::2], keep)
    out = [ev[0]]
    i = 0
    while len(out) < keep and (i < len(od) or i + 1 < len(ev)):
        e = ev[i + 1] if i + 1 < len(ev) else None
        o = od[i] if i < len(od) else None
        if e is None:
            out.append(o)
        elif o is None:
            out.append(e)
        else:
            hi, lo = _ce(e, o)
            out.append(hi)
            out.append(lo)
        i += 1
    return out[:keep]


def _merge_bank_block(top_ref, xn, bank_ref, valid, n):
    chunk = 4 * LANES
    nchunks = (valid + chunk - 1) // chunk
    run = [top_ref[k] for k in range(TOPK)]
    for ci in range(nchunks):
        c0 = ci * chunk
        sc = jax.lax.dot_general(xn, bank_ref[c0:c0 + chunk, :],
                                 (((1,), (1,)), ((), ())))
        groups = []
        for g in range(4):
            gstart = c0 + g * LANES
            if gstart >= valid:
                break
            cg = sc[:, g * LANES:(g + 1) * LANES]
            rem = valid - gstart
            if rem < LANES:
                lane = jax.lax.broadcasted_iota(jnp.int32, (n, LANES), 1)
                cg = jnp.where(lane < rem, cg, -jnp.inf)
            groups.append([cg])
        while len(groups) > 1:
            nxt = []
            for i in range(0, len(groups) - 1, 2):
                nxt.append(_oem(groups[i], groups[i + 1], keep=4))
            if len(groups) % 2:
                nxt.append(groups[-1])
            groups = nxt
        run = _oem(run, groups[0])
    for k in range(TOPK):
        top_ref[k] = run[k]


def _top5_row_stats(top_ref, n):
    comb = jnp.concatenate([top_ref[k] for k in range(TOPK)], axis=1)
    ncand = TOPK * LANES
    lane = jax.lax.broadcasted_iota(jnp.int32, (n, ncand), 1)
    total = jnp.zeros((n, 1), jnp.float32)
    first = None
    for r in range(TOPK):
        m = jnp.max(comb, axis=1, keepdims=True)
        if r == 0:
            first = m
        total = total + m
        idx = jnp.min(jnp.where(comb == m, lane, ncand), axis=1, keepdims=True)
        comb = jnp.where(lane == idx, -jnp.inf, comb)
    return first, total / float(TOPK)


def _body(x_ref, cc_ref, cv_ref, rep_ref, ood_ref, stats_ref, scores_ref,
          xn_ref, rtop_ref, otop_ref, *, n, d, r_blocks, r_last,
          o_blocks, o_last):
    j = pl.program_id(0)

    @pl.when(j == 0)
    def _init():
        x = x_ref[...]
        norm = jnp.sqrt(jnp.sum(x * x, axis=1, keepdims=True))
        xn = x / jnp.maximum(norm, 1e-12)
        xn_ref[...] = xn
        neg = jnp.full((n, LANES), -jnp.inf, jnp.float32)
        for k in range(TOPK):
            rtop_ref[k] = neg
            otop_ref[k] = neg
        cc = cc_ref[...]
        csims = jax.lax.dot_general(xn, cc, (((1,), (1,)), ((), ())))
        stats_ref[:, 0:1] = jnp.max(csims, axis=1, keepdims=True)
        stats_ref[:, 1:2] = jnp.mean(csims, axis=1, keepdims=True)
        inv_v = 1.0 / jnp.clip(cv_ref[...], EPS, None)
        t1 = jax.lax.dot_general(xn * xn, inv_v, (((1,), (1,)), ((), ())))
        t2 = jax.lax.dot_general(xn, cc * inv_v, (((1,), (1,)), ((), ())))
        c2 = jnp.sum(cc * cc * inv_v, axis=1)
        maha = (t1 - 2.0 * t2 + c2[None, :]) * (1.0 / float(d))
        stats_ref[:, 4:5] = -jnp.min(maha, axis=1, keepdims=True)

    xn = xn_ref[...]

    @pl.when(j < r_blocks - 1)
    def _rep_full():
        _merge_bank_block(rtop_ref, xn, rep_ref, BK, n)

    @pl.when(j == r_blocks - 1)
    def _rep_last():
        _merge_bank_block(rtop_ref, xn, rep_ref, r_last, n)

    @pl.when(j < o_blocks - 1)
    def _ood_full():
        _merge_bank_block(otop_ref, xn, ood_ref, BK, n)

    @pl.when(j == o_blocks - 1)
    def _ood_last():
        _merge_bank_block(otop_ref, xn, ood_ref, o_last, n)

    @pl.when(j == r_blocks - 1)
    def _finalize():
        rep_max, rep_mean = _top5_row_stats(rtop_ref, n)
        _, ood_pen = _top5_row_stats(otop_ref, n)
        stats_ref[:, 2:3] = rep_max
        stats_ref[:, 3:4] = rep_mean
        stats_ref[:, 5:6] = ood_pen
        scores_ref[...] = rep_max + stats_ref[:, 4:5] - ood_pen


@jax.jit
def kernel(features, class_centers, class_diag_vars, representatives,
           ood_bank):
    n, d = features.shape
    c = class_centers.shape[0]
    r = representatives.shape[0]
    o = ood_bank.shape[0]
    r_blocks = -(-r // BK)
    o_blocks = -(-o // BK)
    r_last = r - (r_blocks - 1) * BK
    o_last = o - (o_blocks - 1) * BK
    assert r_blocks >= o_blocks

    body = functools.partial(
        _body, n=n, d=d, r_blocks=r_blocks, r_last=r_last,
        o_blocks=o_blocks, o_last=o_last)

    stats, scores = pl.pallas_call(
        body,
        grid=(r_blocks,),
        in_specs=[
            pl.BlockSpec((n, d), lambda j: (0, 0)),
            pl.BlockSpec((c, d), lambda j: (0, 0)),
            pl.BlockSpec((c, d), lambda j: (0, 0)),
            pl.BlockSpec((BK, d), lambda j: (j, 0)),
            pl.BlockSpec((BK, d), lambda j: (jnp.minimum(j, o_blocks - 1), 0)),
        ],
        out_specs=[
            pl.BlockSpec((n, 6), lambda j: (0, 0)),
            pl.BlockSpec((n, 1), lambda j: (0, 0)),
        ],
        out_shape=[
            jax.ShapeDtypeStruct((n, 6), jnp.float32),
            jax.ShapeDtypeStruct((n, 1), jnp.float32),
        ],
        scratch_shapes=[
            pltpu.VMEM((n, d), jnp.float32),
            pltpu.VMEM((TOPK, n, LANES), jnp.float32),
            pltpu.VMEM((TOPK, n, LANES), jnp.float32),
        ],
    )(features.astype(jnp.float32), class_centers, class_diag_vars,
      representatives, ood_bank)
    return stats, scores.reshape(n)


# speedup vs baseline: 1.0196x; 1.0196x over previous
import functools

import jax
import jax.numpy as jnp
from jax.experimental import pallas as pl
from jax.experimental.pallas import tpu as pltpu

EPS = 1e-06
LANES = 128
TOPK = 5
BK = 4096


def _ce(a, b):
    return jnp.maximum(a, b), jnp.minimum(a, b)


def _oem(a, b, keep=TOPK):
    if not a:
        return b[:keep]
    if not b:
        return a[:keep]
    if len(a) == 1 and len(b) == 1:
        hi, lo = _ce(a[0], b[0])
        return [hi, lo][:keep]
    ev = _oem(a[0::2], b[0::2], keep)
    od = _oem(a[1::2], b[1::2], keep)
    out = [ev[0]]
    i = 0
    while len(out) < keep and (i < len(od) or i + 1 < len(ev)):
        e = ev[i + 1] if i + 1 < len(ev) else None
        o = od[i] if i < len(od) else None
        if e is None:
            out.append(o)
        elif o is None:
            out.append(e)
        else:
            hi, lo = _ce(e, o)
            out.append(hi)
            out.append(lo)
        i += 1
    return out[:keep]


def _merge_bank_block(top_ref, xn, bank_ref, valid, n):
    chunk = 4 * LANES
    nchunks = (valid + chunk - 1) // chunk
    lists = []
    for ci in range(nchunks):
        c0 = ci * chunk
        sc = jax.lax.dot_general(xn, bank_ref[c0:c0 + chunk, :],
                                 (((1,), (1,)), ((), ())))
        for g in range(4):
            gstart = c0 + g * LANES
            if gstart >= valid:
                break
            cg = sc[:, g * LANES:(g + 1) * LANES]
            rem = valid - gstart
            if rem < LANES:
                lane = jax.lax.broadcasted_iota(jnp.int32, (n, LANES), 1)
                cg = jnp.where(lane < rem, cg, -jnp.inf)
            lists.append([cg])
    while len(lists) > 1:
        nxt = []
        for i in range(0, len(lists) - 1, 2):
            nxt.append(_oem(lists[i], lists[i + 1]))
        if len(lists) % 2:
            nxt.append(lists[-1])
        lists = nxt
    merged = _oem([top_ref[k] for k in range(TOPK)], lists[0])
    for k in range(TOPK):
        top_ref[k] = merged[k]


def _top5_row_stats(top_ref, n):
    comb = jnp.concatenate([top_ref[k] for k in range(TOPK)], axis=1)
    ncand = TOPK * LANES
    lane = jax.lax.broadcasted_iota(jnp.int32, (n, ncand), 1)
    total = jnp.zeros((n, 1), jnp.float32)
    first = None
    for r in range(TOPK):
        m = jnp.max(comb, axis=1, keepdims=True)
        if r == 0:
            first = m
        total = total + m
        idx = jnp.min(jnp.where(comb == m, lane, ncand), axis=1, keepdims=True)
        comb = jnp.where(lane == idx, -jnp.inf, comb)
    return first, total / float(TOPK)


def _body(x_ref, cc_ref, cv_ref, rep_ref, ood_ref, stats_ref, scores_ref,
          xn_ref, rtop_ref, otop_ref, *, n, d, r_blocks, r_last,
          o_blocks, o_last):
    j = pl.program_id(0)

    @pl.when(j == 0)
    def _init():
        x = x_ref[...]
        norm = jnp.sqrt(jnp.sum(x * x, axis=1, keepdims=True))
        xn = x / jnp.maximum(norm, 1e-12)
        xn_ref[...] = xn
        neg = jnp.full((n, LANES), -jnp.inf, jnp.float32)
        for k in range(TOPK):
            rtop_ref[k] = neg
            otop_ref[k] = neg
        cc = cc_ref[...]
        csims = jax.lax.dot_general(xn, cc, (((1,), (1,)), ((), ())))
        stats_ref[:, 0:1] = jnp.max(csims, axis=1, keepdims=True)
        stats_ref[:, 1:2] = jnp.mean(csims, axis=1, keepdims=True)
        inv_v = 1.0 / jnp.clip(cv_ref[...], EPS, None)
        t1 = jax.lax.dot_general(xn * xn, inv_v, (((1,), (1,)), ((), ())))
        t2 = jax.lax.dot_general(xn, cc * inv_v, (((1,), (1,)), ((), ())))
        c2 = jnp.sum(cc * cc * inv_v, axis=1)
        maha = (t1 - 2.0 * t2 + c2[None, :]) * (1.0 / float(d))
        stats_ref[:, 4:5] = -jnp.min(maha, axis=1, keepdims=True)

    xn = xn_ref[...]

    @pl.when(j < r_blocks - 1)
    def _rep_full():
        _merge_bank_block(rtop_ref, xn, rep_ref, BK, n)

    @pl.when(j == r_blocks - 1)
    def _rep_last():
        _merge_bank_block(rtop_ref, xn, rep_ref, r_last, n)

    @pl.when(j < o_blocks - 1)
    def _ood_full():
        _merge_bank_block(otop_ref, xn, ood_ref, BK, n)

    @pl.when(j == o_blocks - 1)
    def _ood_last():
        _merge_bank_block(otop_ref, xn, ood_ref, o_last, n)

    @pl.when(j == min(o_blocks, r_blocks - 1))
    def _ood_finalize():
        _, ood_pen = _top5_row_stats(otop_ref, n)
        stats_ref[:, 5:6] = ood_pen

    @pl.when(j == r_blocks - 1)
    def _finalize():
        rep_max, rep_mean = _top5_row_stats(rtop_ref, n)
        stats_ref[:, 2:3] = rep_max
        stats_ref[:, 3:4] = rep_mean
        scores_ref[...] = rep_max + stats_ref[:, 4:5] - stats_ref[:, 5:6]


@jax.jit
def kernel(features, class_centers, class_diag_vars, representatives,
           ood_bank):
    n, d = features.shape
    c = class_centers.shape[0]
    r = representatives.shape[0]
    o = ood_bank.shape[0]
    r_blocks = -(-r // BK)
    o_blocks = -(-o // BK)
    r_last = r - (r_blocks - 1) * BK
    o_last = o - (o_blocks - 1) * BK
    assert r_blocks >= o_blocks

    body = functools.partial(
        _body, n=n, d=d, r_blocks=r_blocks, r_last=r_last,
        o_blocks=o_blocks, o_last=o_last)

    stats, scores = pl.pallas_call(
        body,
        grid=(r_blocks,),
        in_specs=[
            pl.BlockSpec((n, d), lambda j: (0, 0)),
            pl.BlockSpec((c, d), lambda j: (0, 0)),
            pl.BlockSpec((c, d), lambda j: (0, 0)),
            pl.BlockSpec((BK, d), lambda j: (j, 0)),
            pl.BlockSpec((BK, d), lambda j: (jnp.minimum(j, o_blocks - 1), 0)),
        ],
        out_specs=[
            pl.BlockSpec((n, 6), lambda j: (0, 0)),
            pl.BlockSpec((n, 1), lambda j: (0, 0)),
        ],
        out_shape=[
            jax.ShapeDtypeStruct((n, 6), jnp.float32),
            jax.ShapeDtypeStruct((n, 1), jnp.float32),
        ],
        scratch_shapes=[
            pltpu.VMEM((n, d), jnp.float32),
            pltpu.VMEM((TOPK, n, LANES), jnp.float32),
            pltpu.VMEM((TOPK, n, LANES), jnp.float32),
        ],
    )(features.astype(jnp.float32), class_centers, class_diag_vars,
      representatives, ood_bank)
    return stats, scores.reshape(n)
